# baseline (speedup 1.0000x reference)
import functools

import jax
import jax.numpy as jnp
from jax import lax
from jax.experimental import pallas as pl
from jax.experimental.pallas import tpu as pltpu
from jax.experimental.pallas import tpu_sc as plsc

N = 10000
E = 320000
F_IN = 128
HID = 16
C = 40
CP = 48

NC = 2
NSUB = 16
NW = NC * NSUB
EW = E // NW
SB = 125
NSTREAM = EW // SB
ROWS_PER_SUB = N // NSUB
NSLOT = 8
NGIN = 4


def _worker_id():
    cid = lax.axis_index("c")
    sid = lax.axis_index("s")
    return cid, sid


NDEG = EW * NC // SB
DWIN = 656
NPAD = 10016


def _rsqrt16(x):
    i = plsc.bitcast(x, jnp.int32)
    i = jnp.full((16,), 0x5F3759DF, jnp.int32) - lax.shift_right_logical(i, 1)
    y = plsc.bitcast(i, jnp.float32)
    xh = x * 0.5
    for _ in range(3):
        y = y * (1.5 - xh * y * y)
    return y


def _layer1_body(dstd_hbm, src_hbm, h1_hbm, ones_hbm, zdeg_hbm, zeros_hbm,
                 agg_hbm, h1n_hbm, d_hbm,
                 dst_v, src_v, deg_v, d_v, d16_v, h1_v, h1n_v, rows_v, ones_v,
                 deg_sh, tab_sh, acc_sh, dsem, ssem, *gsems):
    cid, sid = _worker_id()
    rps = ROWS_PER_SUB

    @pl.when(sid == 0)
    def _():
        pltpu.sync_copy(zdeg_hbm, deg_sh)

    pltpu.sync_copy(zeros_hbm, acc_sh.at[pl.ds(sid * rps, rps)])
    pltpu.sync_copy(ones_hbm, ones_v)
    pltpu.sync_copy(dstd_hbm.at[sid], dst_v)
    pltpu.sync_copy(src_hbm.at[sid, cid], src_v)
    pltpu.sync_copy(h1_hbm.at[sid], h1_v)
    plsc.subcore_barrier()

    def dstep(j, _):
        pltpu.async_copy(ones_v, deg_sh.at[dst_v.at[j]], dsem, add=True)
        return ()

    lax.fori_loop(0, NDEG, dstep, ())

    def ddrain(j, _):
        pltpu.make_async_copy(ones_v, deg_sh.at[dst_v.at[j]], dsem).wait()
        return ()

    lax.fori_loop(0, NDEG, ddrain, ())
    plsc.subcore_barrier()

    pltpu.sync_copy(deg_sh.at[pl.ds(sid * 624, DWIN)], deg_v)

    def nstep(k, _):
        x = deg_v[pl.ds(sid + 16 * k, 16)] + 1.0
        d_v[pl.ds(16 * k, 16)] = _rsqrt16(x)
        return ()

    lax.fori_loop(0, 40, nstep, ())

    def sstep(i5, _):
        for u in range(5):
            i = i5 * 5 + u
            b = plsc.load_gather(d_v, [jnp.full((16,), i, jnp.int32)])
            h1n_v[i] = h1_v[i] * b
            d16_v[i] = b
        return ()

    lax.fori_loop(0, rps // 5, sstep, ())
    pltpu.sync_copy(h1n_v, tab_sh.at[pl.ds(sid * rps, rps)])
    plsc.subcore_barrier()

    def start_g(j, slot):
        pltpu.async_copy(tab_sh.at[src_v.at[j]], rows_v.at[slot], gsems[slot])

    def wait_g(j, slot):
        pltpu.make_async_copy(tab_sh.at[src_v.at[j]], rows_v.at[slot],
                              gsems[slot]).wait()

    def start_s(j, slot):
        pltpu.async_copy(rows_v.at[slot], acc_sh.at[dst_v.at[cid * NSTREAM + j]],
                         ssem, add=True)

    def wait_s_one(j, slot):
        pltpu.make_async_copy(rows_v.at[slot],
                              acc_sh.at[dst_v.at[cid * NSTREAM + j]], ssem).wait()

    for s in range(NGIN):
        start_g(s, s)

    def step(i, _):
        for s in range(NSLOT):
            j = i * NSLOT + s
            wait_g(j, s)
            start_s(j, s)

            @pl.when(j >= NGIN)
            def _():
                wait_s_one(j, s)

            @pl.when(j + NGIN < NSTREAM)
            def _():
                start_g(j + NGIN, (s + NGIN) % NSLOT)

        return ()

    lax.fori_loop(0, NSTREAM // NSLOT, step, ())
    for s in range(NGIN):
        wait_s_one(0, s)
    plsc.subcore_barrier()

    pltpu.sync_copy(acc_sh.at[pl.ds(sid * rps, rps)], agg_hbm.at[cid, sid])

    @pl.when(cid == 0)
    def _():
        pltpu.sync_copy(h1n_v, h1n_hbm.at[sid])
        pltpu.sync_copy(d16_v, d_hbm.at[sid])


_layer1_call = pl.kernel(
    _layer1_body,
    out_type=(jax.ShapeDtypeStruct((NC, NSUB, ROWS_PER_SUB, HID), jnp.float32),
              jax.ShapeDtypeStruct((NSUB, ROWS_PER_SUB, HID), jnp.float32),
              jax.ShapeDtypeStruct((NSUB, ROWS_PER_SUB, HID), jnp.float32)),
    mesh=plsc.VectorSubcoreMesh(core_axis_name="c", subcore_axis_name="s"),
    scratch_types=[
        pltpu.VMEM((NDEG, SB), jnp.int32),
        pltpu.VMEM((NSTREAM, SB), jnp.int32),
        pltpu.VMEM((DWIN,), jnp.float32),
        pltpu.VMEM((640,), jnp.float32),
        pltpu.VMEM((ROWS_PER_SUB, HID), jnp.float32),
        pltpu.VMEM((ROWS_PER_SUB, HID), jnp.float32),
        pltpu.VMEM((ROWS_PER_SUB, HID), jnp.float32),
        pltpu.VMEM((NSLOT, SB, HID), jnp.float32),
        pltpu.VMEM((SB,), jnp.float32),
        pltpu.VMEM_SHARED((NPAD,), jnp.float32),
        pltpu.VMEM_SHARED((N, HID), jnp.float32),
        pltpu.VMEM_SHARED((N, HID), jnp.float32),
        pltpu.SemaphoreType.DMA,
        pltpu.SemaphoreType.DMA,
    ] + [pltpu.SemaphoreType.DMA] * NSLOT,
    compiler_params=pltpu.CompilerParams(use_tc_tiling_on_sc=False,
                                         needs_layout_passes=False),
)


def _spmm_body(width, src_hbm, dst_hbm, table_hbm, zeros_hbm, out_hbm,
               src_v, dst_v, rows_v, acc_sh, ssem, *gsems):
    cid, sid = _worker_id()
    wid = sid * NC + cid
    rps = ROWS_PER_SUB
    pltpu.sync_copy(zeros_hbm, acc_sh.at[pl.ds(sid * rps, rps)])
    pltpu.sync_copy(src_hbm.at[wid], src_v)
    pltpu.sync_copy(dst_hbm.at[wid], dst_v)
    plsc.subcore_barrier()

    def start_g(j, slot):
        pltpu.async_copy(table_hbm.at[src_v.at[j]], rows_v.at[slot], gsems[slot])

    def wait_g(j, slot):
        pltpu.make_async_copy(table_hbm.at[src_v.at[j]], rows_v.at[slot],
                              gsems[slot]).wait()

    def start_s(j, slot):
        pltpu.async_copy(rows_v.at[slot], acc_sh.at[dst_v.at[j]], ssem, add=True)

    def wait_s_one(j, slot):
        pltpu.make_async_copy(rows_v.at[slot], acc_sh.at[dst_v.at[j]], ssem).wait()

    for s in range(NGIN):
        start_g(s, s)

    def step(i, _):
        for s in range(NSLOT):
            j = i * NSLOT + s
            wait_g(j, s)
            start_s(j, s)

            @pl.when(j >= NGIN)
            def _():
                wait_s_one(j, s)

            @pl.when(j + NGIN < NSTREAM)
            def _():
                start_g(j + NGIN, (s + NGIN) % NSLOT)

        return ()

    lax.fori_loop(0, NSTREAM // NSLOT, step, ())
    for s in range(NGIN):
        wait_s_one(0, s)
    plsc.subcore_barrier()
    pltpu.sync_copy(acc_sh.at[pl.ds(sid * rps, rps)], out_hbm.at[cid, sid])


def _make_spmm(width):
    return pl.kernel(
        functools.partial(_spmm_body, width),
        out_type=jax.ShapeDtypeStruct((NC, NSUB, ROWS_PER_SUB, width), jnp.float32),
        mesh=plsc.VectorSubcoreMesh(core_axis_name="c", subcore_axis_name="s"),
        scratch_types=[
            pltpu.VMEM((NSTREAM, SB), jnp.int32),
            pltpu.VMEM((NSTREAM, SB), jnp.int32),
            pltpu.VMEM((NSLOT, SB, width), jnp.float32),
            pltpu.VMEM_SHARED((N, width), jnp.float32),
            pltpu.SemaphoreType.DMA,
        ] + [pltpu.SemaphoreType.DMA] * NSLOT,
        compiler_params=pltpu.CompilerParams(use_tc_tiling_on_sc=False),
    )


_spmm48 = _make_spmm(CP)


def _tc1_body(x_ref, w1_ref, h1_ref):
    h1_ref[...] = jnp.dot(x_ref[...], w1_ref[...],
                          preferred_element_type=jnp.float32)


_tc1 = pl.pallas_call(
    _tc1_body,
    out_shape=jax.ShapeDtypeStruct((N, HID), jnp.float32),
)


def _tc2_body(agg_ref, h1n_ref, dinv_ref, b1_ref, w2_ref, h2n_ref):
    dinv = dinv_ref[...]
    z1 = dinv * (agg_ref[0] + agg_ref[1] + h1n_ref[...]) + b1_ref[...]
    z1 = jnp.maximum(z1, 0.0)
    h2 = jnp.dot(z1, w2_ref[...], preferred_element_type=jnp.float32)
    d48 = jnp.concatenate([dinv, dinv, dinv], axis=1)
    h2n_ref[...] = h2 * d48


_tc2 = pl.pallas_call(
    _tc2_body,
    out_shape=jax.ShapeDtypeStruct((N, CP), jnp.float32),
)


def _tc3_body(agg_ref, h2n_ref, dinv_ref, b2_ref, out_ref):
    dinv = dinv_ref[...]
    d48 = jnp.concatenate([dinv, dinv, dinv], axis=1)
    z2 = d48 * (agg_ref[0] + agg_ref[1] + h2n_ref[...]) + b2_ref[...]
    z = z2[:, :C]
    m = jnp.max(z, axis=1, keepdims=True)
    e = jnp.exp(z - m)
    lse = jnp.log(jnp.sum(e, axis=1, keepdims=True))
    out_ref[...] = z - m - lse


_tc3 = pl.pallas_call(
    _tc3_body,
    out_shape=jax.ShapeDtypeStruct((N, C), jnp.float32),
)


def kernel(x, edge_index, W1, b1, W2, b2):
    src_sp = edge_index[0].reshape(NSUB, NC, NSTREAM, SB)
    dst_d = edge_index[1].reshape(NSUB, NDEG, SB)
    src = edge_index[0].reshape(NW, NSTREAM, SB)
    dst = edge_index[1].reshape(NW, NSTREAM, SB)

    ones1 = jnp.ones((SB,), jnp.float32)
    zdeg = jnp.zeros((NPAD,), jnp.float32)
    zeros16 = jnp.zeros((ROWS_PER_SUB, HID), jnp.float32)
    zeros48 = jnp.zeros((ROWS_PER_SUB, CP), jnp.float32)

    h1 = _tc1(x, W1)
    agg1, h1n, d = _layer1_call(dst_d, src_sp,
                                h1.reshape(NSUB, ROWS_PER_SUB, HID),
                                ones1, zdeg, zeros16)
    agg1 = agg1.reshape(NC, N, HID)
    h1n = h1n.reshape(N, HID)
    dinv = d.reshape(N, HID)
    W2p = jnp.pad(W2, ((0, 0), (0, CP - C)))
    b2p = jnp.pad(b2, (0, CP - C))
    h2n = _tc2(agg1, h1n, dinv, b1.reshape(1, HID), W2p)
    agg2 = _spmm48(src, dst, h2n, zeros48).reshape(NC, N, CP)
    out = _tc3(agg2, h2n, dinv, b2p.reshape(1, CP))
    return out


# speedup vs baseline: 1.1106x; 1.1106x over previous
import functools

import jax
import jax.numpy as jnp
from jax import lax
from jax.experimental import pallas as pl
from jax.experimental.pallas import tpu as pltpu
from jax.experimental.pallas import tpu_sc as plsc

N = 10000
E = 320000
F_IN = 128
HID = 16
C = 40
CP = 48

NC = 2
NSUB = 16
NW = NC * NSUB
EW = E // NW
SB = 125
NSTREAM = EW // SB
ROWS_PER_SUB = N // NSUB


def _worker_id():
    cid = lax.axis_index("c")
    sid = lax.axis_index("s")
    return cid, sid


def _deg_body(dst_hbm, ones_hbm, zeros_hbm, out_hbm, dst_v, ones_v, acc_sh, dsem):
    cid, sid = _worker_id()
    wid = sid * NC + cid

    @pl.when(sid == 0)
    def _():
        pltpu.sync_copy(zeros_hbm, acc_sh)

    pltpu.sync_copy(ones_hbm, ones_v)
    pltpu.sync_copy(dst_hbm.at[wid], dst_v)
    plsc.subcore_barrier()

    def step(j, _):
        pltpu.async_copy(ones_v, acc_sh.at[dst_v.at[j]], dsem, add=True)
        return ()

    lax.fori_loop(0, NSTREAM, step, ())

    def drain(j, _):
        pltpu.make_async_copy(ones_v, acc_sh.at[dst_v.at[j]], dsem).wait()
        return ()

    lax.fori_loop(0, NSTREAM, drain, ())
    plsc.subcore_barrier()

    @pl.when(sid == 0)
    def _():
        pltpu.sync_copy(acc_sh, out_hbm.at[cid])


_deg_call = pl.kernel(
    _deg_body,
    out_type=jax.ShapeDtypeStruct((NC, N), jnp.float32),
    mesh=plsc.VectorSubcoreMesh(core_axis_name="c", subcore_axis_name="s"),
    scratch_types=[
        pltpu.VMEM((NSTREAM, SB), jnp.int32),
        pltpu.VMEM((SB,), jnp.float32),
        pltpu.VMEM_SHARED((N,), jnp.float32),
        pltpu.SemaphoreType.DMA,
    ],
    compiler_params=pltpu.CompilerParams(use_tc_tiling_on_sc=False),
)


def _spmm_body(width, nslot, ngin, src_hbm, dst_hbm, table_hbm, zeros_hbm,
               out_hbm, src_v, dst_v, rows_v, acc_sh, ssem, *gsems):
    cid, sid = _worker_id()
    wid = sid * NC + cid
    rps = ROWS_PER_SUB
    pltpu.sync_copy(zeros_hbm, acc_sh.at[pl.ds(sid * rps, rps)])
    pltpu.sync_copy(src_hbm.at[wid], src_v)
    pltpu.sync_copy(dst_hbm.at[wid], dst_v)
    plsc.subcore_barrier()

    def start_g(j, slot):
        pltpu.async_copy(table_hbm.at[src_v.at[j]], rows_v.at[slot], gsems[slot])

    def wait_g(j, slot):
        pltpu.make_async_copy(table_hbm.at[src_v.at[j]], rows_v.at[slot],
                              gsems[slot]).wait()

    def start_s(j, slot):
        pltpu.async_copy(rows_v.at[slot], acc_sh.at[dst_v.at[j]], ssem, add=True)

    def wait_s_one(j, slot):
        pltpu.make_async_copy(rows_v.at[slot], acc_sh.at[dst_v.at[j]], ssem).wait()

    for s in range(ngin):
        start_g(s, s)

    def step(i, _):
        for s in range(nslot):
            j = i * nslot + s
            wait_g(j, s)
            start_s(j, s)

            @pl.when(j >= ngin)
            def _():
                wait_s_one(j, s)

            @pl.when(j + ngin < NSTREAM)
            def _():
                start_g(j + ngin, (s + ngin) % nslot)

        return ()

    lax.fori_loop(0, NSTREAM // nslot, step, ())
    for s in range(ngin):
        wait_s_one(0, s)
    plsc.subcore_barrier()
    pltpu.sync_copy(acc_sh.at[pl.ds(sid * rps, rps)], out_hbm.at[cid, sid])


def _make_spmm(width, nslot, ngin):
    return pl.kernel(
        functools.partial(_spmm_body, width, nslot, ngin),
        out_type=jax.ShapeDtypeStruct((NC, NSUB, ROWS_PER_SUB, width), jnp.float32),
        mesh=plsc.VectorSubcoreMesh(core_axis_name="c", subcore_axis_name="s"),
        scratch_types=[
            pltpu.VMEM((NSTREAM, SB), jnp.int32),
            pltpu.VMEM((NSTREAM, SB), jnp.int32),
            pltpu.VMEM((nslot, SB, width), jnp.float32),
            pltpu.VMEM_SHARED((N, width), jnp.float32),
            pltpu.SemaphoreType.DMA,
        ] + [pltpu.SemaphoreType.DMA] * nslot,
        compiler_params=pltpu.CompilerParams(use_tc_tiling_on_sc=False),
    )


_spmm16 = _make_spmm(HID, 16, 8)
_spmm48 = _make_spmm(CP, 10, 5)


def _tc1_body(x_ref, w1_ref, degp_ref, h1n_ref, dinv_ref):
    deg = degp_ref[0, :] + degp_ref[1, :] + 1.0
    dinv = lax.rsqrt(deg)[:, None]
    h1 = jnp.dot(x_ref[...], w1_ref[...], preferred_element_type=jnp.float32)
    dinv_ref[...] = dinv
    h1n_ref[...] = h1 * dinv


_tc1 = pl.pallas_call(
    _tc1_body,
    out_shape=(jax.ShapeDtypeStruct((N, HID), jnp.float32),
               jax.ShapeDtypeStruct((N, 1), jnp.float32)),
)


def _tc2_body(agg_ref, h1n_ref, dinv_ref, b1_ref, w2_ref, h2n_ref):
    dinv = dinv_ref[...]
    z1 = dinv * (agg_ref[0] + agg_ref[1] + h1n_ref[...]) + b1_ref[...]
    z1 = jnp.maximum(z1, 0.0)
    h2 = jnp.dot(z1, w2_ref[...], preferred_element_type=jnp.float32)
    h2n_ref[...] = h2 * dinv


_tc2 = pl.pallas_call(
    _tc2_body,
    out_shape=jax.ShapeDtypeStruct((N, CP), jnp.float32),
)


def _tc3_body(agg_ref, h2n_ref, dinv_ref, b2_ref, out_ref):
    z2 = dinv_ref[...] * (agg_ref[0] + agg_ref[1] + h2n_ref[...]) + b2_ref[...]
    z = z2[:, :C]
    m = jnp.max(z, axis=1, keepdims=True)
    e = jnp.exp(z - m)
    lse = jnp.log(jnp.sum(e, axis=1, keepdims=True))
    out_ref[...] = z - m - lse


_tc3 = pl.pallas_call(
    _tc3_body,
    out_shape=jax.ShapeDtypeStruct((N, C), jnp.float32),
)


def kernel(x, edge_index, W1, b1, W2, b2):
    src = edge_index[0].reshape(NW, NSTREAM, SB)
    dst = edge_index[1].reshape(NW, NSTREAM, SB)

    ones1 = jnp.ones((SB,), jnp.float32)
    zerosN = jnp.zeros((N,), jnp.float32)
    zeros16 = jnp.zeros((ROWS_PER_SUB, HID), jnp.float32)
    zeros48 = jnp.zeros((ROWS_PER_SUB, CP), jnp.float32)

    degp = _deg_call(dst, ones1, zerosN)
    h1n, dinv = _tc1(x, W1, degp)
    agg1 = _spmm16(src, dst, h1n, zeros16).reshape(NC, N, HID)
    W2p = jnp.pad(W2, ((0, 0), (0, CP - C)))
    b2p = jnp.pad(b2, (0, CP - C))
    h2n = _tc2(agg1, h1n, dinv, b1.reshape(1, HID), W2p)
    agg2 = _spmm48(src, dst, h2n, zeros48).reshape(NC, N, CP)
    out = _tc3(agg2, h2n, dinv, b2p.reshape(1, CP))
    return out


# speedup vs baseline: 1.1418x; 1.0281x over previous
import functools

import jax
import jax.numpy as jnp
from jax import lax
from jax.experimental import pallas as pl
from jax.experimental.pallas import tpu as pltpu
from jax.experimental.pallas import tpu_sc as plsc

N = 10000
E = 320000
F_IN = 128
HID = 16
C = 40
CP = 48

NC = 2
NSUB = 16
NW = NC * NSUB
EW = E // NW
SB = 125
NSTREAM = EW // SB
ROWS_PER_SUB = N // NSUB


def _worker_id():
    cid = lax.axis_index("c")
    sid = lax.axis_index("s")
    return cid, sid


def _deg_body(dst_hbm, ones_hbm, zeros_hbm, out_hbm, dst_v, ones_v, acc_sh, dsem):
    cid, sid = _worker_id()
    wid = sid * NC + cid

    @pl.when(sid == 0)
    def _():
        pltpu.sync_copy(zeros_hbm, acc_sh)

    pltpu.sync_copy(ones_hbm, ones_v)
    pltpu.sync_copy(dst_hbm.at[wid], dst_v)
    plsc.subcore_barrier()

    def step(j, _):
        pltpu.async_copy(ones_v, acc_sh.at[dst_v.at[j]], dsem, add=True)
        return ()

    lax.fori_loop(0, NSTREAM, step, ())

    def drain(j, _):
        pltpu.make_async_copy(ones_v, acc_sh.at[dst_v.at[j]], dsem).wait()
        return ()

    lax.fori_loop(0, NSTREAM, drain, ())
    plsc.subcore_barrier()

    @pl.when(sid == 0)
    def _():
        pltpu.sync_copy(acc_sh, out_hbm.at[cid])


_deg_call = pl.kernel(
    _deg_body,
    out_type=jax.ShapeDtypeStruct((NC, N), jnp.float32),
    mesh=plsc.VectorSubcoreMesh(core_axis_name="c", subcore_axis_name="s"),
    scratch_types=[
        pltpu.VMEM((NSTREAM, SB), jnp.int32),
        pltpu.VMEM((SB,), jnp.float32),
        pltpu.VMEM_SHARED((N,), jnp.float32),
        pltpu.SemaphoreType.DMA,
    ],
    compiler_params=pltpu.CompilerParams(use_tc_tiling_on_sc=False),
)


def _spmm_body(width, nslot, ngin, src_hbm, dst_hbm, table_hbm, zeros_hbm,
               out_hbm, src_v, dst_v, rows_v, acc_sh, ssem, *gsems):
    cid, sid = _worker_id()
    wid = sid * NC + cid
    rps = ROWS_PER_SUB
    pltpu.sync_copy(zeros_hbm, acc_sh.at[pl.ds(sid * rps, rps)])
    pltpu.sync_copy(src_hbm.at[wid], src_v)
    pltpu.sync_copy(dst_hbm.at[wid], dst_v)
    plsc.subcore_barrier()

    def start_g(j, slot):
        pltpu.async_copy(table_hbm.at[src_v.at[j]], rows_v.at[slot], gsems[slot])

    def wait_g(j, slot):
        pltpu.make_async_copy(table_hbm.at[src_v.at[j]], rows_v.at[slot],
                              gsems[slot]).wait()

    def start_s(j, slot):
        pltpu.async_copy(rows_v.at[slot], acc_sh.at[dst_v.at[j]], ssem, add=True)

    def wait_s_one(j, slot):
        pltpu.make_async_copy(rows_v.at[slot], acc_sh.at[dst_v.at[j]], ssem).wait()

    for s in range(ngin):
        start_g(s, s)

    def step(i, _):
        for s in range(nslot):
            j = i * nslot + s
            wait_g(j, s)
            start_s(j, s)

            @pl.when(j >= ngin)
            def _():
                wait_s_one(j, s)

            @pl.when(j + ngin < NSTREAM)
            def _():
                start_g(j + ngin, (s + ngin) % nslot)

        return ()

    lax.fori_loop(0, NSTREAM // nslot, step, ())
    for s in range(ngin):
        wait_s_one(0, s)
    plsc.subcore_barrier()
    pltpu.sync_copy(acc_sh.at[pl.ds(sid * rps, rps)], out_hbm.at[cid, sid])


def _make_spmm(width, nslot, ngin):
    return pl.kernel(
        functools.partial(_spmm_body, width, nslot, ngin),
        out_type=jax.ShapeDtypeStruct((NC, NSUB, ROWS_PER_SUB, width), jnp.float32),
        mesh=plsc.VectorSubcoreMesh(core_axis_name="c", subcore_axis_name="s"),
        scratch_types=[
            pltpu.VMEM((NSTREAM, SB), jnp.int32),
            pltpu.VMEM((NSTREAM, SB), jnp.int32),
            pltpu.VMEM((nslot, SB, width), jnp.float32),
            pltpu.VMEM_SHARED((N, width), jnp.float32),
            pltpu.SemaphoreType.DMA,
        ] + [pltpu.SemaphoreType.DMA] * nslot,
        compiler_params=pltpu.CompilerParams(use_tc_tiling_on_sc=False),
    )


_spmm16 = _make_spmm(HID, 16, 8)
_spmm40 = _make_spmm(C, 16, 8)


def _tc1_body(x_ref, w1_ref, degp_ref, h1n_ref, dinv_ref):
    deg = degp_ref[0, :] + degp_ref[1, :] + 1.0
    dinv = lax.rsqrt(deg)[:, None]
    h1 = jnp.dot(x_ref[...], w1_ref[...], preferred_element_type=jnp.float32)
    dinv_ref[...] = dinv
    h1n_ref[...] = h1 * dinv


_tc1 = pl.pallas_call(
    _tc1_body,
    out_shape=(jax.ShapeDtypeStruct((N, HID), jnp.float32),
               jax.ShapeDtypeStruct((N, 1), jnp.float32)),
)


def _tc2_body(agg_ref, h1n_ref, dinv_ref, b1_ref, w2_ref, h2n_ref):
    dinv = dinv_ref[...]
    z1 = dinv * (agg_ref[0] + agg_ref[1] + h1n_ref[...]) + b1_ref[...]
    z1 = jnp.maximum(z1, 0.0)
    h2 = jnp.dot(z1, w2_ref[...], preferred_element_type=jnp.float32)
    h2n_ref[...] = h2 * dinv


_tc2 = pl.pallas_call(
    _tc2_body,
    out_shape=jax.ShapeDtypeStruct((N, C), jnp.float32),
)


def _tc3_body(agg_ref, h2n_ref, dinv_ref, b2_ref, out_ref):
    z = dinv_ref[...] * (agg_ref[0] + agg_ref[1] + h2n_ref[...]) + b2_ref[...]
    m = jnp.max(z, axis=1, keepdims=True)
    e = jnp.exp(z - m)
    lse = jnp.log(jnp.sum(e, axis=1, keepdims=True))
    out_ref[...] = z - m - lse


_tc3 = pl.pallas_call(
    _tc3_body,
    out_shape=jax.ShapeDtypeStruct((N, C), jnp.float32),
)


def kernel(x, edge_index, W1, b1, W2, b2):
    src = edge_index[0].reshape(NW, NSTREAM, SB)
    dst = edge_index[1].reshape(NW, NSTREAM, SB)

    ones1 = jnp.ones((SB,), jnp.float32)
    zerosN = jnp.zeros((N,), jnp.float32)
    zeros16 = jnp.zeros((ROWS_PER_SUB, HID), jnp.float32)
    zeros40 = jnp.zeros((ROWS_PER_SUB, C), jnp.float32)

    degp = _deg_call(dst, ones1, zerosN)
    h1n, dinv = _tc1(x, W1, degp)
    agg1 = _spmm16(src, dst, h1n, zeros16).reshape(NC, N, HID)
    h2n = _tc2(agg1, h1n, dinv, b1.reshape(1, HID), W2)
    agg2 = _spmm40(src, dst, h2n, zeros40).reshape(NC, N, C)
    out = _tc3(agg2, h2n, dinv, b2.reshape(1, C))
    return out


# speedup vs baseline: 1.1447x; 1.0026x over previous
import functools

import jax
import jax.numpy as jnp
from jax import lax
from jax.experimental import pallas as pl
from jax.experimental.pallas import tpu as pltpu
from jax.experimental.pallas import tpu_sc as plsc

N = 10000
E = 320000
F_IN = 128
HID = 16
C = 40
CP = 48

NC = 2
NSUB = 16
NW = NC * NSUB
EW = E // NW
SB = 125
NSTREAM = EW // SB
ROWS_PER_SUB = N // NSUB


def _worker_id():
    cid = lax.axis_index("c")
    sid = lax.axis_index("s")
    return cid, sid


def _deg_body(dst_hbm, ones_hbm, zeros_hbm, out_hbm, dst_v, ones_v, acc_sh, dsem):
    cid, sid = _worker_id()
    wid = sid * NC + cid

    @pl.when(sid == 0)
    def _():
        pltpu.sync_copy(zeros_hbm, acc_sh)

    pltpu.sync_copy(ones_hbm, ones_v)
    pltpu.sync_copy(dst_hbm.at[wid], dst_v)
    plsc.subcore_barrier()

    def step(j, _):
        pltpu.async_copy(ones_v, acc_sh.at[dst_v.at[j]], dsem, add=True)
        return ()

    lax.fori_loop(0, NSTREAM, step, ())

    def drain(j, _):
        pltpu.make_async_copy(ones_v, acc_sh.at[dst_v.at[j]], dsem).wait()
        return ()

    lax.fori_loop(0, NSTREAM, drain, ())
    plsc.subcore_barrier()

    @pl.when(sid == 0)
    def _():
        pltpu.sync_copy(acc_sh, out_hbm.at[cid])


_deg_call = pl.kernel(
    _deg_body,
    out_type=jax.ShapeDtypeStruct((NC, N), jnp.float32),
    mesh=plsc.VectorSubcoreMesh(core_axis_name="c", subcore_axis_name="s"),
    scratch_types=[
        pltpu.VMEM((NSTREAM, SB), jnp.int32),
        pltpu.VMEM((SB,), jnp.float32),
        pltpu.VMEM_SHARED((N,), jnp.float32),
        pltpu.SemaphoreType.DMA,
    ],
    compiler_params=pltpu.CompilerParams(use_tc_tiling_on_sc=False),
)


def _spmm_body(width, nslot, ngin, src_hbm, dst_hbm, table_hbm, zeros_hbm,
               out_hbm, src_v, dst_v, rows_v, acc_sh, ssem, *gsems):
    cid, sid = _worker_id()
    wid = sid * NC + cid
    rps = ROWS_PER_SUB
    pltpu.sync_copy(zeros_hbm, acc_sh.at[pl.ds(sid * rps, rps)])
    pltpu.sync_copy(src_hbm.at[wid], src_v)
    pltpu.sync_copy(dst_hbm.at[wid], dst_v)
    plsc.subcore_barrier()

    def start_g(j, slot):
        pltpu.async_copy(table_hbm.at[src_v.at[j]], rows_v.at[slot], gsems[slot])

    def wait_g(j, slot):
        pltpu.make_async_copy(table_hbm.at[src_v.at[j]], rows_v.at[slot],
                              gsems[slot]).wait()

    def start_s(j, slot):
        pltpu.async_copy(rows_v.at[slot], acc_sh.at[dst_v.at[j]], ssem, add=True)

    def wait_s_one(j, slot):
        pltpu.make_async_copy(rows_v.at[slot], acc_sh.at[dst_v.at[j]], ssem).wait()

    for s in range(ngin):
        start_g(s, s)

    def step(i, _):
        for s in range(nslot):
            j = i * nslot + s
            wait_g(j, s)
            start_s(j, s)

            @pl.when(j >= ngin)
            def _():
                wait_s_one(j, s)

            @pl.when(j + ngin < NSTREAM)
            def _():
                start_g(j + ngin, (s + ngin) % nslot)

        return ()

    lax.fori_loop(0, NSTREAM // nslot, step, ())
    for s in range(ngin):
        wait_s_one(0, s)
    plsc.subcore_barrier()
    pltpu.sync_copy(acc_sh.at[pl.ds(sid * rps, rps)], out_hbm.at[cid, sid])


def _make_spmm(width, nslot, ngin):
    return pl.kernel(
        functools.partial(_spmm_body, width, nslot, ngin),
        out_type=jax.ShapeDtypeStruct((NC, NSUB, ROWS_PER_SUB, width), jnp.float32),
        mesh=plsc.VectorSubcoreMesh(core_axis_name="c", subcore_axis_name="s"),
        scratch_types=[
            pltpu.VMEM((NSTREAM, SB), jnp.int32),
            pltpu.VMEM((NSTREAM, SB), jnp.int32),
            pltpu.VMEM((nslot, SB, width), jnp.float32),
            pltpu.VMEM_SHARED((N, width), jnp.float32),
            pltpu.SemaphoreType.DMA,
        ] + [pltpu.SemaphoreType.DMA] * nslot,
        compiler_params=pltpu.CompilerParams(use_tc_tiling_on_sc=False),
    )


_spmm16 = _make_spmm(HID, 20, 10)
_spmm40 = _make_spmm(C, 16, 8)


def _tc1_body(x_ref, w1_ref, degp_ref, h1n_ref, dinv_ref):
    deg = degp_ref[0, :] + degp_ref[1, :] + 1.0
    dinv = lax.rsqrt(deg)[:, None]
    h1 = jnp.dot(x_ref[...], w1_ref[...], preferred_element_type=jnp.float32)
    dinv_ref[...] = dinv
    h1n_ref[...] = h1 * dinv


_tc1 = pl.pallas_call(
    _tc1_body,
    out_shape=(jax.ShapeDtypeStruct((N, HID), jnp.float32),
               jax.ShapeDtypeStruct((N, 1), jnp.float32)),
)


def _tc2_body(agg_ref, h1n_ref, dinv_ref, b1_ref, w2_ref, h2n_ref):
    dinv = dinv_ref[...]
    z1 = dinv * (agg_ref[0] + agg_ref[1] + h1n_ref[...]) + b1_ref[...]
    z1 = jnp.maximum(z1, 0.0)
    h2 = jnp.dot(z1, w2_ref[...], preferred_element_type=jnp.float32)
    h2n_ref[...] = h2 * dinv


_tc2 = pl.pallas_call(
    _tc2_body,
    out_shape=jax.ShapeDtypeStruct((N, C), jnp.float32),
)


def _tc3_body(agg_ref, h2n_ref, dinv_ref, b2_ref, out_ref):
    z = dinv_ref[...] * (agg_ref[0] + agg_ref[1] + h2n_ref[...]) + b2_ref[...]
    m = jnp.max(z, axis=1, keepdims=True)
    e = jnp.exp(z - m)
    lse = jnp.log(jnp.sum(e, axis=1, keepdims=True))
    out_ref[...] = z - m - lse


_tc3 = pl.pallas_call(
    _tc3_body,
    out_shape=jax.ShapeDtypeStruct((N, C), jnp.float32),
)


def kernel(x, edge_index, W1, b1, W2, b2):
    src = edge_index[0].reshape(NW, NSTREAM, SB)
    dst = edge_index[1].reshape(NW, NSTREAM, SB)

    ones1 = jnp.ones((SB,), jnp.float32)
    zerosN = jnp.zeros((N,), jnp.float32)
    zeros16 = jnp.zeros((ROWS_PER_SUB, HID), jnp.float32)
    zeros40 = jnp.zeros((ROWS_PER_SUB, C), jnp.float32)

    degp = _deg_call(dst, ones1, zerosN)
    h1n, dinv = _tc1(x, W1, degp)
    agg1 = _spmm16(src, dst, h1n, zeros16).reshape(NC, N, HID)
    h2n = _tc2(agg1, h1n, dinv, b1.reshape(1, HID), W2)
    agg2 = _spmm40(src, dst, h2n, zeros40).reshape(NC, N, C)
    out = _tc3(agg2, h2n, dinv, b2.reshape(1, C))
    return out
